# Initial kernel scaffold; baseline (speedup 1.0000x reference)
import jax
import jax.numpy as jnp
from jax.experimental import pallas as pl


def kernel(points, vertices, vertex_normals):
    raise NotImplementedError("write your pallas kernel here")


# speedup vs baseline: 41.9553x; 41.9553x over previous
import functools

import jax
import jax.numpy as jnp
from jax.experimental import pallas as pl

_KNN_K = 60
_NSUP = 7


def _radii_kernel(v8_ref, vt_ref, sr_ref, *, tile):
    a = v8_ref[...]
    vt = vt_ref[...]
    ab = jnp.dot(a, vt, preferred_element_type=jnp.float32)
    aa = jnp.sum(a * a, axis=1, keepdims=True)
    bb = jnp.sum(vt * vt, axis=0, keepdims=True)
    d2 = aa + bb - 2.0 * ab
    acc = jnp.zeros((tile, 1), jnp.float32)
    first = jnp.zeros((tile, 1), jnp.float32)
    cur = d2
    for i in range(_NSUP):
        m = jnp.min(cur, axis=1, keepdims=True)
        acc = acc + m
        if i == 0:
            first = m
        cur = jnp.where(cur == m, jnp.float32(jnp.inf), cur)
    sr_ref[...] = (acc - first) * jnp.float32(2.0 / 6.0)


def _sdf_kernel(p8_ref, vt_ref, vnt_ref, sr_ref, out_ref, *, n_iter):
    p = p8_ref[...]
    vt = vt_ref[...]
    vnt = vnt_ref[...]
    ab = jnp.dot(p, vt, preferred_element_type=jnp.float32)
    aa = jnp.sum(p * p, axis=1, keepdims=True)
    bb = jnp.sum(vt * vt, axis=0, keepdims=True)
    d2 = aa + bb - 2.0 * ab

    bits = jax.lax.bitcast_convert_type(jnp.maximum(d2, 0.0), jnp.int32)
    hi = jnp.max(bits, axis=1, keepdims=True)
    lo = jnp.zeros_like(hi)

    def body(_, carry):
        lo, hi = carry
        mid = lo + jax.lax.shift_right_logical(hi - lo, 1)
        cnt = jnp.sum((bits <= mid).astype(jnp.int32), axis=1, keepdims=True)
        ge = cnt >= _KNN_K
        return jnp.where(ge, lo, mid + 1), jnp.where(ge, mid, hi)

    _, kth = jax.lax.fori_loop(0, n_iter, body, (lo, hi))
    sel = bits <= kth

    dx = p[:, 0:1] - vt[0:1, :]
    dy = p[:, 1:2] - vt[1:2, :]
    dz = p[:, 2:3] - vt[2:3, :]
    d2c = dx * dx + dy * dy + dz * dz
    dot = vnt[0:1, :] * dx + vnt[1:2, :] * dy + vnt[2:3, :] * dz
    sr = sr_ref[...]
    q = 1.0 - d2c / sr
    phi = (q * q) ** 2
    w = jnp.where(d2c < sr, phi, jnp.float32(1e-18))
    w = jnp.where(sel, w, jnp.float32(0.0))
    num = jnp.sum(w * dot, axis=1, keepdims=True)
    den = jnp.sum(w, axis=1, keepdims=True)
    sdf = num / den
    acc = jnp.sum(sdf * sdf, axis=0, keepdims=True)

    @pl.when(pl.program_id(0) == 0)
    def _():
        out_ref[...] = jnp.zeros((1, 1), jnp.float32)

    out_ref[...] += acc


def _pick_tile(n):
    for t in (256, 128, 64, 32, 16, 8):
        if n % t == 0:
            return t
    return n


def kernel(points, vertices, vertex_normals):
    n, num_p, _ = points.shape
    num_v = vertices.shape[1]
    p = points.reshape(num_p, 3)
    v = vertices.reshape(num_v, 3)
    vn = vertex_normals.reshape(num_v, 3)

    vp = ((num_v + 127) // 128) * 128
    pad = vp - num_v
    if pad:
        v = jnp.concatenate([v, jnp.full((pad, 3), 1e17, jnp.float32)], axis=0)
        vn = jnp.concatenate([vn, jnp.zeros((pad, 3), jnp.float32)], axis=0)
    zcol = jnp.zeros((5, vp), jnp.float32)
    vt = jnp.concatenate([v.T, zcol], axis=0)
    vnt = jnp.concatenate([vn.T, zcol], axis=0)
    p8 = jnp.concatenate([p, jnp.zeros((num_p, 5), jnp.float32)], axis=1)

    tile_a = _pick_tile(vp)
    sr = pl.pallas_call(
        functools.partial(_radii_kernel, tile=tile_a),
        grid=(vp // tile_a,),
        in_specs=[
            pl.BlockSpec((tile_a, 8), lambda i: (i, 0)),
            pl.BlockSpec((8, vp), lambda i: (0, 0)),
        ],
        out_specs=pl.BlockSpec((tile_a, 1), lambda i: (i, 0)),
        out_shape=jax.ShapeDtypeStruct((vp, 1), jnp.float32),
    )(vt.T, vt)

    sr_row = sr.reshape(1, vp)

    tile_b = _pick_tile(num_p)
    out = pl.pallas_call(
        functools.partial(_sdf_kernel, n_iter=31),
        grid=(num_p // tile_b,),
        in_specs=[
            pl.BlockSpec((tile_b, 8), lambda i: (i, 0)),
            pl.BlockSpec((8, vp), lambda i: (0, 0)),
            pl.BlockSpec((8, vp), lambda i: (0, 0)),
            pl.BlockSpec((1, vp), lambda i: (0, 0)),
        ],
        out_specs=pl.BlockSpec((1, 1), lambda i: (0, 0)),
        out_shape=jax.ShapeDtypeStruct((1, 1), jnp.float32),
    )(p8, vt, vnt, sr_row)

    return out.reshape(n)


# speedup vs baseline: 80.2187x; 1.9120x over previous
import functools

import jax
import jax.numpy as jnp
from jax.experimental import pallas as pl

_KNN_K = 60
_NSUP = 7


def _lane_topL(d2, tile, vp, depth):
    regs = [jnp.full((tile, 128), jnp.inf, jnp.float32) for _ in range(depth)]
    for c in range(vp // 128):
        cur = d2[:, c * 128:(c + 1) * 128]
        for j in range(depth):
            lo = jnp.minimum(regs[j], cur)
            cur = jnp.maximum(regs[j], cur)
            regs[j] = lo
    return jnp.concatenate(regs, axis=1)


def _kth_smallest(comp, tile, k, n_iter):
    bits = jax.lax.bitcast_convert_type(comp, jnp.int32)
    hi = jnp.max(bits, axis=1, keepdims=True)
    lo = jnp.zeros((tile, 1), jnp.int32)

    def body(_, carry):
        lo, hi = carry
        mid = lo + jax.lax.shift_right_logical(hi - lo, 1)
        cnt = jnp.sum((bits <= mid).astype(jnp.int32), axis=1, keepdims=True)
        ge = cnt >= k
        return jnp.where(ge, lo, mid + 1), jnp.where(ge, mid, hi)

    _, kth = jax.lax.fori_loop(0, n_iter, body, (lo, hi))
    return jax.lax.bitcast_convert_type(kth, jnp.float32)


def _radii_kernel(v8_ref, vt_ref, sr_ref, *, tile, vp):
    a = v8_ref[...]
    vt = vt_ref[...]
    ab = jnp.dot(a, vt, preferred_element_type=jnp.float32)
    aa = jnp.sum(a * a, axis=1, keepdims=True)
    bb = jnp.sum(vt * vt, axis=0, keepdims=True)
    d2 = aa + bb - 2.0 * ab
    comp = _lane_topL(d2, tile, vp, 4)
    acc = jnp.zeros((tile, 1), jnp.float32)
    first = jnp.zeros((tile, 1), jnp.float32)
    cur = comp
    for i in range(_NSUP):
        m = jnp.min(cur, axis=1, keepdims=True)
        acc = acc + m
        if i == 0:
            first = m
        cur = jnp.where(cur == m, jnp.float32(jnp.inf), cur)
    sr_ref[...] = (acc - first) * jnp.float32(2.0 / 6.0)


def _sdf_kernel(p8_ref, vt_ref, vnt_ref, sr_ref, out_ref, *, tile, vp, n_iter):
    p = p8_ref[...]
    vt = vt_ref[...]
    vnt = vnt_ref[...]
    ab = jnp.dot(p, vt, preferred_element_type=jnp.float32)
    aa = jnp.sum(p * p, axis=1, keepdims=True)
    bb = jnp.sum(vt * vt, axis=0, keepdims=True)
    dpos = jnp.maximum(aa + bb - 2.0 * ab, 0.0)

    comp = _lane_topL(dpos, tile, vp, 8)
    kth = _kth_smallest(comp, tile, _KNN_K, n_iter)
    sel = dpos <= kth

    dx = p[:, 0:1] - vt[0:1, :]
    dy = p[:, 1:2] - vt[1:2, :]
    dz = p[:, 2:3] - vt[2:3, :]
    d2c = dx * dx + dy * dy + dz * dz
    c = jnp.sum(vt * vnt, axis=0, keepdims=True)
    dot = jnp.dot(p, vnt, preferred_element_type=jnp.float32) - c
    sr = sr_ref[...]
    q = 1.0 - d2c / sr
    phi = (q * q) ** 2
    w = jnp.where(d2c < sr, phi, jnp.float32(1e-18))
    w = jnp.where(sel, w, jnp.float32(0.0))
    num = jnp.sum(w * dot, axis=1, keepdims=True)
    den = jnp.sum(w, axis=1, keepdims=True)
    sdf = num / den
    acc = jnp.sum(sdf * sdf, axis=0, keepdims=True)

    @pl.when(pl.program_id(0) == 0)
    def _():
        out_ref[...] = jnp.zeros((1, 1), jnp.float32)

    out_ref[...] += acc


def _pick_tile(n):
    for t in (256, 128, 64, 32, 16, 8):
        if n % t == 0:
            return t
    return n


def kernel(points, vertices, vertex_normals):
    n, num_p, _ = points.shape
    num_v = vertices.shape[1]
    p = points.reshape(num_p, 3)
    v = vertices.reshape(num_v, 3)
    vn = vertex_normals.reshape(num_v, 3)

    vp = ((num_v + 127) // 128) * 128
    pad = vp - num_v
    if pad:
        v = jnp.concatenate([v, jnp.full((pad, 3), 1e17, jnp.float32)], axis=0)
        vn = jnp.concatenate([vn, jnp.zeros((pad, 3), jnp.float32)], axis=0)
    zcol = jnp.zeros((5, vp), jnp.float32)
    vt = jnp.concatenate([v.T, zcol], axis=0)
    vnt = jnp.concatenate([vn.T, zcol], axis=0)
    p8 = jnp.concatenate([p, jnp.zeros((num_p, 5), jnp.float32)], axis=1)

    tile_a = _pick_tile(vp)
    sr = pl.pallas_call(
        functools.partial(_radii_kernel, tile=tile_a, vp=vp),
        grid=(vp // tile_a,),
        in_specs=[
            pl.BlockSpec((tile_a, 8), lambda i: (i, 0)),
            pl.BlockSpec((8, vp), lambda i: (0, 0)),
        ],
        out_specs=pl.BlockSpec((tile_a, 1), lambda i: (i, 0)),
        out_shape=jax.ShapeDtypeStruct((vp, 1), jnp.float32),
    )(vt.T, vt)

    sr_row = sr.reshape(1, vp)

    tile_b = _pick_tile(num_p)
    out = pl.pallas_call(
        functools.partial(_sdf_kernel, tile=tile_b, vp=vp, n_iter=31),
        grid=(num_p // tile_b,),
        in_specs=[
            pl.BlockSpec((tile_b, 8), lambda i: (i, 0)),
            pl.BlockSpec((8, vp), lambda i: (0, 0)),
            pl.BlockSpec((8, vp), lambda i: (0, 0)),
            pl.BlockSpec((1, vp), lambda i: (0, 0)),
        ],
        out_specs=pl.BlockSpec((1, 1), lambda i: (0, 0)),
        out_shape=jax.ShapeDtypeStruct((1, 1), jnp.float32),
    )(p8, vt, vnt, sr_row)

    return out.reshape(n)
